# baseline (speedup 1.0000x reference)
import functools

import jax
import jax.numpy as jnp
from jax import lax
from jax.experimental import pallas as pl
from jax.experimental.pallas import tpu as pltpu
from jax.experimental.pallas import tpu_sc as plsc

_TOP_K = 8
_BLK = 2048
_E = 64
_TOKENS = 16384
_NWORKERS = 32
_TPW = _TOKENS // _NWORKERS


def _logits_body(x_ref, w_ref, out_ref):
    x = x_ref[...]
    w = w_ref[...]
    out_ref[...] = jax.lax.dot_general(
        x, w, (((1,), (1,)), ((), ())),
        preferred_element_type=jnp.float32)


def _logits(x, weight):
    tokens = x.shape[0]
    return pl.pallas_call(
        _logits_body,
        grid=(tokens // _BLK,),
        in_specs=[
            pl.BlockSpec((_BLK, x.shape[1]), lambda i: (i, 0)),
            pl.BlockSpec(weight.shape, lambda i: (0, 0)),
        ],
        out_specs=pl.BlockSpec((_BLK, _E), lambda i: (i, 0)),
        out_shape=jax.ShapeDtypeStruct((tokens, _E), jnp.float32),
        compiler_params=pltpu.CompilerParams(
            dimension_semantics=("parallel",)),
    )(x, weight)


def _topk_sc_body(logits_hbm, idx_hbm, wgt_hbm, sc_v, oi_v, ow_v):
    wid = lax.axis_index("s") * 2 + lax.axis_index("c")
    base = wid * (_TPW * _E)
    pltpu.sync_copy(logits_hbm.at[pl.ds(base, _TPW * _E)], sc_v)

    lane = lax.iota(jnp.int32, 16)
    lo8 = lane < 8

    def token_body(t, _):
        off = t * _E
        ks = [sc_v[pl.ds(off + j * 16, 16)] for j in range(4)]
        vs = [lane + j * 16 for j in range(4)]
        srt = [plsc.sort_key_val(ks[j], vs[j], descending=True)
               for j in range(4)]
        def merge(a, b):
            ck = jnp.where(lo8, a[0], jnp.flip(b[0], 0))
            cv = jnp.where(lo8, a[1], jnp.flip(b[1], 0))
            return plsc.sort_key_val(ck, cv, descending=True)
        fk, fv = merge(merge(srt[0], srt[1]), merge(srt[2], srt[3]))
        m = jnp.max(fk)
        e = jnp.where(lo8, jnp.exp(fk - m), 0.0)
        w = e / jnp.sum(e)
        ow_v[pl.ds(t * _TOP_K, 16)] = w
        oi_v[pl.ds(t * _TOP_K, 16)] = fv
        return _

    lax.fori_loop(0, _TPW, token_body, None)

    obase = wid * (_TPW * _TOP_K)
    pltpu.sync_copy(oi_v.at[pl.ds(0, _TPW * _TOP_K)],
                    idx_hbm.at[pl.ds(obase, _TPW * _TOP_K)])
    pltpu.sync_copy(ow_v.at[pl.ds(0, _TPW * _TOP_K)],
                    wgt_hbm.at[pl.ds(obase, _TPW * _TOP_K)])


def _topk_sc(logits_flat):
    n_out = _TOKENS * _TOP_K
    pad = _TPW * _TOP_K + 8
    fn = functools.partial(
        pl.kernel, _topk_sc_body,
        mesh=plsc.VectorSubcoreMesh(core_axis_name="c", subcore_axis_name="s"),
        out_type=[
            jax.ShapeDtypeStruct((n_out,), jnp.int32),
            jax.ShapeDtypeStruct((n_out,), jnp.float32),
        ],
        scratch_types=[
            pltpu.VMEM((_TPW * _E,), jnp.float32),
            pltpu.VMEM((pad,), jnp.int32),
            pltpu.VMEM((pad,), jnp.float32),
        ],
        compiler_params=pltpu.CompilerParams(needs_layout_passes=False),
    )()
    return fn(logits_flat)


@jax.jit
def _gate(x, weight):
    logits = _logits(x, weight)
    idx_flat, wgt_flat = _topk_sc(logits.reshape(-1))
    return (idx_flat.reshape(_TOKENS, _TOP_K),
            wgt_flat.reshape(_TOKENS, _TOP_K))


def kernel(hidden_states, weight):
    bsz, seq_len, h = hidden_states.shape
    x = hidden_states.reshape(-1, h)
    topk_idx, topk_weight = _gate(x, weight)
    return (topk_idx, topk_weight, jnp.float32(0.0))


# speedup vs baseline: 1.1992x; 1.1992x over previous
import functools

import jax
import jax.numpy as jnp
from jax import lax
from jax.experimental import pallas as pl
from jax.experimental.pallas import tpu as pltpu
from jax.experimental.pallas import tpu_sc as plsc

_TOP_K = 8
_BLK = 2048
_E = 64
_TOKENS = 16384
_NCHUNKS = 4
_CHUNK = _TOKENS // _NCHUNKS
_NWORKERS = 32
_TPW = _CHUNK // _NWORKERS


def _logits_body(x_ref, w_ref, out_ref):
    x = x_ref[...]
    w = w_ref[...]
    logits_t = jax.lax.dot_general(
        w, x, (((1,), (1,)), ((), ())),
        preferred_element_type=jnp.float32)
    out_ref[...] = logits_t.T


def _logits_chunk(x, weight, chunk):
    blocks = _CHUNK // _BLK
    return pl.pallas_call(
        _logits_body,
        grid=(blocks,),
        in_specs=[
            pl.BlockSpec((_BLK, x.shape[1]),
                         lambda i, c=chunk: (c * blocks + i, 0)),
            pl.BlockSpec(weight.shape, lambda i: (0, 0)),
        ],
        out_specs=pl.BlockSpec((_BLK, _E), lambda i: (i, 0)),
        out_shape=jax.ShapeDtypeStruct((_CHUNK, _E), jnp.float32),
    )(x, weight)


def _topk_sc_body(logits_hbm, idx_hbm, wgt_hbm, sc_v, oi_v, ow_v):
    wid = lax.axis_index("s") * 2 + lax.axis_index("c")
    base = wid * (_TPW * _E)
    pltpu.sync_copy(logits_hbm.at[pl.ds(base, _TPW * _E)], sc_v)

    lane = lax.iota(jnp.int32, 16)
    lo8 = lane < 8

    def token_body(t, carry):
        off = t * _E
        ks = [sc_v[pl.ds(off + j * 16, 16)] for j in range(4)]
        vs = [lane + j * 16 for j in range(4)]
        srt = [plsc.sort_key_val(ks[j], vs[j], descending=True)
               for j in range(4)]

        def merge(a, b):
            ck = jnp.where(lo8, a[0], jnp.flip(b[0], 0))
            cv = jnp.where(lo8, a[1], jnp.flip(b[1], 0))
            return plsc.sort_key_val(ck, cv, descending=True)

        fk, fv = merge(merge(srt[0], srt[1]), merge(srt[2], srt[3]))
        m = jnp.max(fk)
        e = jnp.where(lo8, jnp.exp(fk - m), 0.0)
        w = e / jnp.sum(e)
        ow_v[pl.ds(t * _TOP_K, 16)] = w
        oi_v[pl.ds(t * _TOP_K, 16)] = fv
        return carry

    lax.fori_loop(0, _TPW, token_body, None)

    obase = wid * (_TPW * _TOP_K)
    pltpu.sync_copy(oi_v.at[pl.ds(0, _TPW * _TOP_K)],
                    idx_hbm.at[pl.ds(obase, _TPW * _TOP_K)])
    pltpu.sync_copy(ow_v.at[pl.ds(0, _TPW * _TOP_K)],
                    wgt_hbm.at[pl.ds(obase, _TPW * _TOP_K)])


def _topk_sc(logits_flat):
    n_out = _CHUNK * _TOP_K
    pad = _TPW * _TOP_K + 8
    fn = functools.partial(
        pl.kernel, _topk_sc_body,
        mesh=plsc.VectorSubcoreMesh(core_axis_name="c", subcore_axis_name="s"),
        out_type=[
            jax.ShapeDtypeStruct((n_out,), jnp.int32),
            jax.ShapeDtypeStruct((n_out,), jnp.float32),
        ],
        scratch_types=[
            pltpu.VMEM((_TPW * _E,), jnp.float32),
            pltpu.VMEM((pad,), jnp.int32),
            pltpu.VMEM((pad,), jnp.float32),
        ],
        compiler_params=pltpu.CompilerParams(needs_layout_passes=False),
    )()
    return fn(logits_flat)


@jax.jit
def _gate(x, weight):
    idxs = []
    wgts = []
    for c in range(_NCHUNKS):
        logits_c = _logits_chunk(x, weight, c)
        idx_c, wgt_c = _topk_sc(logits_c.reshape(-1))
        idxs.append(idx_c.reshape(_CHUNK, _TOP_K))
        wgts.append(wgt_c.reshape(_CHUNK, _TOP_K))
    return jnp.concatenate(idxs, axis=0), jnp.concatenate(wgts, axis=0)


def kernel(hidden_states, weight):
    bsz, seq_len, h = hidden_states.shape
    x = hidden_states.reshape(-1, h)
    topk_idx, topk_weight = _gate(x, weight)
    return (topk_idx, topk_weight, jnp.float32(0.0))


# speedup vs baseline: 1.2172x; 1.0150x over previous
import functools

import jax
import jax.numpy as jnp
from jax import lax
from jax.experimental import pallas as pl
from jax.experimental.pallas import tpu as pltpu
from jax.experimental.pallas import tpu_sc as plsc

_TOP_K = 8
_BLK = 2048
_E = 64
_TOKENS = 16384
_NCHUNKS = 4
_CHUNK = _TOKENS // _NCHUNKS
_NWORKERS = 32
_TPW = _CHUNK // _NWORKERS


def _logits_body(x_ref, w_ref, out_ref):
    x = x_ref[...]
    w = w_ref[...]
    logits_t = jax.lax.dot_general(
        w, x, (((1,), (1,)), ((), ())),
        preferred_element_type=jnp.float32)
    out_ref[...] = logits_t.T


def _logits_chunk(x, weight, chunk):
    blocks = _CHUNK // _BLK
    return pl.pallas_call(
        _logits_body,
        grid=(blocks,),
        in_specs=[
            pl.BlockSpec((_BLK, x.shape[1]),
                         lambda i, c=chunk: (c * blocks + i, 0)),
            pl.BlockSpec(weight.shape, lambda i: (0, 0)),
        ],
        out_specs=pl.BlockSpec((_BLK, _E), lambda i: (i, 0)),
        out_shape=jax.ShapeDtypeStruct((_CHUNK, _E), jnp.float32),
    )(x, weight)


def _topk_sc_body(logits_hbm, idx_hbm, wgt_hbm, sc_v, oi_v, ow_v):
    wid = lax.axis_index("s") * 2 + lax.axis_index("c")
    base = wid * (_TPW * _E)
    pltpu.sync_copy(logits_hbm.at[pl.ds(base, _TPW * _E)], sc_v)

    lane = lax.iota(jnp.int32, 16)
    lo8 = lane < 8

    @plsc.parallel_loop(0, _TPW, step=1, unroll=4)
    def token_body(t):
        off = t * _E
        ks = [sc_v[pl.ds(off + j * 16, 16)] for j in range(4)]
        vs = [lane + j * 16 for j in range(4)]
        srt = [plsc.sort_key_val(ks[j], vs[j], descending=True)
               for j in range(4)]

        def merge(a, b):
            ck = jnp.where(lo8, a[0], jnp.flip(b[0], 0))
            cv = jnp.where(lo8, a[1], jnp.flip(b[1], 0))
            return plsc.sort_key_val(ck, cv, descending=True)

        fk, fv = merge(merge(srt[0], srt[1]), merge(srt[2], srt[3]))
        m = jnp.max(fk)
        e = jnp.where(lo8, jnp.exp(fk - m), 0.0)
        w = e / jnp.sum(e)
        plsc.store_compressed(ow_v.at[pl.ds(t * _TOP_K, 16)], w, mask=lo8)
        plsc.store_compressed(oi_v.at[pl.ds(t * _TOP_K, 16)], fv, mask=lo8)

    obase = wid * (_TPW * _TOP_K)
    pltpu.sync_copy(oi_v.at[pl.ds(0, _TPW * _TOP_K)],
                    idx_hbm.at[pl.ds(obase, _TPW * _TOP_K)])
    pltpu.sync_copy(ow_v.at[pl.ds(0, _TPW * _TOP_K)],
                    wgt_hbm.at[pl.ds(obase, _TPW * _TOP_K)])


def _topk_sc(logits_flat):
    n_out = _CHUNK * _TOP_K
    pad = _TPW * _TOP_K + 8
    fn = functools.partial(
        pl.kernel, _topk_sc_body,
        mesh=plsc.VectorSubcoreMesh(core_axis_name="c", subcore_axis_name="s"),
        out_type=[
            jax.ShapeDtypeStruct((n_out,), jnp.int32),
            jax.ShapeDtypeStruct((n_out,), jnp.float32),
        ],
        scratch_types=[
            pltpu.VMEM((_TPW * _E,), jnp.float32),
            pltpu.VMEM((pad,), jnp.int32),
            pltpu.VMEM((pad,), jnp.float32),
        ],
        compiler_params=pltpu.CompilerParams(needs_layout_passes=False),
    )()
    return fn(logits_flat)


@jax.jit
def _gate(x, weight):
    idxs = []
    wgts = []
    for c in range(_NCHUNKS):
        logits_c = _logits_chunk(x, weight, c)
        idx_c, wgt_c = _topk_sc(logits_c.reshape(-1))
        idxs.append(idx_c.reshape(_CHUNK, _TOP_K))
        wgts.append(wgt_c.reshape(_CHUNK, _TOP_K))
    return jnp.concatenate(idxs, axis=0), jnp.concatenate(wgts, axis=0)


def kernel(hidden_states, weight):
    bsz, seq_len, h = hidden_states.shape
    x = hidden_states.reshape(-1, h)
    topk_idx, topk_weight = _gate(x, weight)
    return (topk_idx, topk_weight, jnp.float32(0.0))


# speedup vs baseline: 1.2495x; 1.0265x over previous
import functools

import jax
import jax.numpy as jnp
from jax import lax
from jax.experimental import pallas as pl
from jax.experimental.pallas import tpu as pltpu
from jax.experimental.pallas import tpu_sc as plsc

_TOP_K = 8
_BLK = 2048
_E = 64
_TOKENS = 16384
_NCHUNKS = 1
_CHUNK = _TOKENS // _NCHUNKS
_NWORKERS = 32
_TPW = _CHUNK // _NWORKERS


def _logits_body(x_ref, w_ref, out_ref):
    x = x_ref[...]
    w = w_ref[...]
    logits_t = jax.lax.dot_general(
        w, x, (((1,), (1,)), ((), ())),
        preferred_element_type=jnp.float32)
    out_ref[...] = logits_t.T


def _logits_chunk(x, weight, chunk):
    blocks = _CHUNK // _BLK
    return pl.pallas_call(
        _logits_body,
        grid=(blocks,),
        in_specs=[
            pl.BlockSpec((_BLK, x.shape[1]),
                         lambda i, c=chunk: (c * blocks + i, 0)),
            pl.BlockSpec(weight.shape, lambda i: (0, 0)),
        ],
        out_specs=pl.BlockSpec((_BLK, _E), lambda i: (i, 0)),
        out_shape=jax.ShapeDtypeStruct((_CHUNK, _E), jnp.float32),
    )(x, weight)


def _topk_sc_body(logits_hbm, idx_hbm, wgt_hbm, sc_v, oi_v, ow_v):
    wid = lax.axis_index("s") * 2 + lax.axis_index("c")
    base = wid * (_TPW * _E)
    pltpu.sync_copy(logits_hbm.at[pl.ds(base, _TPW * _E)], sc_v)

    lane = lax.iota(jnp.int32, 16)
    lo8 = lane < 8

    @plsc.parallel_loop(0, _TPW, step=1, unroll=4)
    def token_body(t):
        off = t * _E
        ks = [sc_v[pl.ds(off + j * 16, 16)] for j in range(4)]
        vs = [lane + j * 16 for j in range(4)]
        srt = [plsc.sort_key_val(ks[j], vs[j], descending=True)
               for j in range(4)]

        def merge(a, b):
            ck = jnp.where(lo8, a[0], jnp.flip(b[0], 0))
            cv = jnp.where(lo8, a[1], jnp.flip(b[1], 0))
            return plsc.sort_key_val(ck, cv, descending=True)

        fk, fv = merge(merge(srt[0], srt[1]), merge(srt[2], srt[3]))
        m = jnp.max(fk)
        e = jnp.where(lo8, jnp.exp(fk - m), 0.0)
        w = e / jnp.sum(e)
        plsc.store_compressed(ow_v.at[pl.ds(t * _TOP_K, 16)], w, mask=lo8)
        plsc.store_compressed(oi_v.at[pl.ds(t * _TOP_K, 16)], fv, mask=lo8)

    obase = wid * (_TPW * _TOP_K)
    pltpu.sync_copy(oi_v.at[pl.ds(0, _TPW * _TOP_K)],
                    idx_hbm.at[pl.ds(obase, _TPW * _TOP_K)])
    pltpu.sync_copy(ow_v.at[pl.ds(0, _TPW * _TOP_K)],
                    wgt_hbm.at[pl.ds(obase, _TPW * _TOP_K)])


def _topk_sc(logits_flat):
    n_out = _CHUNK * _TOP_K
    pad = _TPW * _TOP_K + 8
    fn = functools.partial(
        pl.kernel, _topk_sc_body,
        mesh=plsc.VectorSubcoreMesh(core_axis_name="c", subcore_axis_name="s"),
        out_type=[
            jax.ShapeDtypeStruct((n_out,), jnp.int32),
            jax.ShapeDtypeStruct((n_out,), jnp.float32),
        ],
        scratch_types=[
            pltpu.VMEM((_TPW * _E,), jnp.float32),
            pltpu.VMEM((pad,), jnp.int32),
            pltpu.VMEM((pad,), jnp.float32),
        ],
        compiler_params=pltpu.CompilerParams(needs_layout_passes=False),
    )()
    return fn(logits_flat)


@jax.jit
def _gate(x, weight):
    idxs = []
    wgts = []
    for c in range(_NCHUNKS):
        logits_c = _logits_chunk(x, weight, c)
        idx_c, wgt_c = _topk_sc(logits_c.reshape(-1))
        idxs.append(idx_c.reshape(_CHUNK, _TOP_K))
        wgts.append(wgt_c.reshape(_CHUNK, _TOP_K))
    return jnp.concatenate(idxs, axis=0), jnp.concatenate(wgts, axis=0)


def kernel(hidden_states, weight):
    bsz, seq_len, h = hidden_states.shape
    x = hidden_states.reshape(-1, h)
    topk_idx, topk_weight = _gate(x, weight)
    return (topk_idx, topk_weight, jnp.float32(0.0))


# speedup vs baseline: 1.2645x; 1.0120x over previous
import functools

import jax
import jax.numpy as jnp
from jax import lax
from jax.experimental import pallas as pl
from jax.experimental.pallas import tpu as pltpu
from jax.experimental.pallas import tpu_sc as plsc

_TOP_K = 8
_BLK = 2048
_E = 64
_TOKENS = 16384
_NCHUNKS = 1
_CHUNK = _TOKENS // _NCHUNKS
_NWORKERS = 32
_TPW = _CHUNK // _NWORKERS


def _logits_body(x_ref, w_ref, out_ref):
    x = x_ref[...]
    w = w_ref[...]
    logits_t = jax.lax.dot_general(
        w, x, (((1,), (1,)), ((), ())),
        preferred_element_type=jnp.float32)
    out_ref[...] = logits_t.T


def _logits_chunk(x, weight, chunk):
    blocks = _CHUNK // _BLK
    return pl.pallas_call(
        _logits_body,
        grid=(blocks,),
        in_specs=[
            pl.BlockSpec((_BLK, x.shape[1]),
                         lambda i, c=chunk: (c * blocks + i, 0)),
            pl.BlockSpec(weight.shape, lambda i: (0, 0)),
        ],
        out_specs=pl.BlockSpec((_BLK, _E), lambda i: (i, 0)),
        out_shape=jax.ShapeDtypeStruct((_CHUNK, _E), jnp.float32),
    )(x, weight)


def _topk_sc_body(logits_hbm, idx_hbm, wgt_hbm, sc_v, oi_v, ow_v):
    wid = lax.axis_index("s") * 2 + lax.axis_index("c")
    base = wid * (_TPW * _E)
    pltpu.sync_copy(logits_hbm.at[pl.ds(base, _TPW * _E)], sc_v)

    lane = lax.iota(jnp.int32, 16)
    lo8 = lane < 8
    shift8 = jnp.maximum(lane - 8, 0)

    def topk_one(off):
        ks = [sc_v[pl.ds(off + j * 16, 16)] for j in range(4)]
        vs = [lane + j * 16 for j in range(4)]
        srt = [plsc.sort_key_val(ks[j], vs[j], descending=True)
               for j in range(4)]

        def merge(a, b):
            ck = jnp.where(lo8, a[0], jnp.flip(b[0], 0))
            cv = jnp.where(lo8, a[1], jnp.flip(b[1], 0))
            return plsc.sort_key_val(ck, cv, descending=True)

        fk, fv = merge(merge(srt[0], srt[1]), merge(srt[2], srt[3]))
        m = jnp.max(fk)
        e = jnp.where(lo8, jnp.exp(fk - m), 0.0)
        w = e / jnp.sum(e)
        return fv, w

    @plsc.parallel_loop(0, _TPW, step=2, unroll=4)
    def token_body(t):
        ia, wa = topk_one(t * _E)
        ib, wb = topk_one((t + 1) * _E)
        ipair = jnp.where(lo8, ia, ib.at[shift8].get(mode="promise_in_bounds"))
        wpair = jnp.where(lo8, wa, wb.at[shift8].get(mode="promise_in_bounds"))
        oi_v[pl.ds(t * _TOP_K, 16)] = ipair
        ow_v[pl.ds(t * _TOP_K, 16)] = wpair

    obase = wid * (_TPW * _TOP_K)
    pltpu.sync_copy(oi_v.at[pl.ds(0, _TPW * _TOP_K)],
                    idx_hbm.at[pl.ds(obase, _TPW * _TOP_K)])
    pltpu.sync_copy(ow_v.at[pl.ds(0, _TPW * _TOP_K)],
                    wgt_hbm.at[pl.ds(obase, _TPW * _TOP_K)])


def _topk_sc(logits_flat):
    n_out = _CHUNK * _TOP_K
    pad = _TPW * _TOP_K + 8
    fn = functools.partial(
        pl.kernel, _topk_sc_body,
        mesh=plsc.VectorSubcoreMesh(core_axis_name="c", subcore_axis_name="s"),
        out_type=[
            jax.ShapeDtypeStruct((n_out,), jnp.int32),
            jax.ShapeDtypeStruct((n_out,), jnp.float32),
        ],
        scratch_types=[
            pltpu.VMEM((_TPW * _E,), jnp.float32),
            pltpu.VMEM((pad,), jnp.int32),
            pltpu.VMEM((pad,), jnp.float32),
        ],
        compiler_params=pltpu.CompilerParams(needs_layout_passes=False),
    )()
    return fn(logits_flat)


@jax.jit
def _gate(x, weight):
    idxs = []
    wgts = []
    for c in range(_NCHUNKS):
        logits_c = _logits_chunk(x, weight, c)
        idx_c, wgt_c = _topk_sc(logits_c.reshape(-1))
        idxs.append(idx_c.reshape(_CHUNK, _TOP_K))
        wgts.append(wgt_c.reshape(_CHUNK, _TOP_K))
    return jnp.concatenate(idxs, axis=0), jnp.concatenate(wgts, axis=0)


def kernel(hidden_states, weight):
    bsz, seq_len, h = hidden_states.shape
    x = hidden_states.reshape(-1, h)
    topk_idx, topk_weight = _gate(x, weight)
    return (topk_idx, topk_weight, jnp.float32(0.0))
